# Initial kernel scaffold; baseline (speedup 1.0000x reference)
import jax
import jax.numpy as jnp
from jax.experimental import pallas as pl


def kernel(feature_2d, fm_w, fm_b, kw0, kb0, kw1, kb1, kw2, kb2, kw3, kb3, qkv_w, qkv_b, merge_w, merge_b, mlp_w1, mlp_b1, mlp_w2, mlp_b2, fp_w, fp_b, cw1, cb1, bn_g, bn_b, cw2, cb2, keypoints):
    raise NotImplementedError("write your pallas kernel here")


# speedup vs baseline: 1.0144x; 1.0144x over previous
import functools

import jax
import jax.numpy as jnp
from jax import lax
from jax.experimental import pallas as pl
from jax.experimental.pallas import tpu as pltpu
from jax.experimental.pallas import tpu_sc as plsc

F32 = jnp.float32
BF16 = jnp.bfloat16

H = W = 224
HW = H * W
N = 1024
D = 256
HEADS = 4
DH = D // HEADS
C1 = 64
C2 = 128
PAD = 256
FR = HW + 2 * PAD
FH = FR // 2
NTAP = 9 * N
NC, NS = 2, 16
RTOT = FR // 4
PASSES = 3
PR = RTOT // PASSES
SPR = PR + 8
STRIPE = PR // NS
TROWS = NTAP // NS
TCH = 288
TROWS = NTAP // NS
GPW = N // (NC * NS)
NT = 8
TP = HW // NT
WIN = TP + 2 * PAD


@functools.cache
def _gather_sc_k():
    @functools.partial(
        pl.kernel,
        out_type=jax.ShapeDtypeStruct((N, 128), F32),
        mesh=plsc.VectorSubcoreMesh(core_axis_name="c",
                                    subcore_axis_name="s"),
        scratch_types=[
            pltpu.VMEM((GPW,), jnp.int32),
            pltpu.VMEM((GPW, 128), F32),
            pltpu.SemaphoreType.DMA,
        ],
    )
    def k(table_hbm, idx_hbm, out_hbm, idx_v, rows_v, sem):
        wid = lax.axis_index("s") * NC + lax.axis_index("c")
        base = wid * GPW
        pltpu.sync_copy(idx_hbm.at[pl.ds(base, GPW)], idx_v)
        pltpu.async_copy(table_hbm.at[idx_v], rows_v, sem).wait()
        pltpu.sync_copy(rows_v, out_hbm.at[pl.ds(base, GPW)])

    return k


def _gather_sc(ft, idx):
    return _gather_sc_k()(ft, idx)


@functools.cache
def _scatter_sc_k():
    @functools.partial(
        pl.kernel,
        out_type=jax.ShapeDtypeStruct((NC, RTOT, 128), F32),
        mesh=plsc.VectorSubcoreMesh(core_axis_name="c",
                                    subcore_axis_name="s"),
        scratch_types=[
            pltpu.VMEM((TCH, 128), F32),
            pltpu.VMEM((TCH,), jnp.int32),
            pltpu.VMEM((STRIPE, 128), F32),
            pltpu.VMEM_SHARED((SPR, 128), F32),
        ],
    )
    def k(u4_hbm, pos_a_hbm, pos_b_hbm, pos_c_hbm, zr_hbm, f_out,
          ubuf, idxv, obuf, shared):
        c = lax.axis_index("c")
        s = lax.axis_index("s")
        for p, pos_hbm in enumerate((pos_a_hbm, pos_b_hbm, pos_c_hbm)):
            pltpu.sync_copy(zr_hbm, obuf)
            pltpu.sync_copy(obuf, shared.at[pl.ds(s * STRIPE, STRIPE)])
            plsc.subcore_barrier()
            for t in range(TROWS // TCH):
                base = s * TROWS + t * TCH
                pltpu.sync_copy(u4_hbm.at[c, pl.ds(base, TCH)], ubuf)
                pltpu.sync_copy(pos_hbm.at[pl.ds(base, TCH)], idxv)
                pltpu.sync_copy(ubuf, shared.at[idxv], add=True)
            plsc.subcore_barrier()
            pltpu.sync_copy(shared.at[pl.ds(s * STRIPE, STRIPE)], obuf)
            pltpu.sync_copy(
                obuf, f_out.at[c, pl.ds(p * PR + s * STRIPE, STRIPE)])
            plsc.subcore_barrier()

    return k


def _scatter_sc(u4, pos3, zr):
    return _scatter_sc_k()(u4, pos3[0], pos3[1], pos3[2], zr)


def _mm(a, b, dims):
    return lax.dot_general(a.astype(BF16), b.astype(BF16), (dims, ((), ())),
                           preferred_element_type=F32)


def _mm32(a, b, dims):
    return lax.dot_general(a, b, (dims, ((), ())),
                           precision=lax.Precision.HIGHEST,
                           preferred_element_type=F32)


def _mmnt(a, b):
    return _mm32(a, b, ((1,), (0,)))


def _gnn_body(g_ref, kpn_ref, fm_w_ref, fm_b_ref,
              kw0_ref, kb0_ref, kw1_ref, kb1_ref, kw2_ref, kb2_ref,
              kw3_ref, kb3_ref, qkv_w_ref, qkv_b_ref, merge_w_ref,
              merge_b_ref, mlp_w1_ref, mlp_b1_ref, mlp_w2_ref, mlp_b2_ref,
              fp_w_ref, fp_b_ref, w1cat_ref, desc0_out, u_out):
    g = g_ref[...]
    gf = _mm32(fm_w_ref[...], g, ((1,), (1,))) + fm_b_ref[...]
    x = kpn_ref[...]
    kw0 = kw0_ref[...]
    x = kw0[:, 0:1] * x[0:1] + kw0[:, 1:2] * x[1:2] + kb0_ref[...]
    x = jnp.maximum(x, 0.0)
    x = jnp.maximum(_mmnt(kw1_ref[...], x) + kb1_ref[...], 0.0)
    x = jnp.maximum(_mmnt(kw2_ref[...], x) + kb2_ref[...], 0.0)
    x = _mmnt(kw3_ref[...], x) + kb3_ref[...]
    desc = gf + x
    for l in range(4):
        q = _mmnt(qkv_w_ref[l, 0], desc) + qkv_b_ref[l, 0]
        k = _mmnt(qkv_w_ref[l, 1], desc) + qkv_b_ref[l, 1]
        v = _mmnt(qkv_w_ref[l, 2], desc) + qkv_b_ref[l, 2]
        msgs = []
        for h in range(HEADS):
            qh = q[h * DH:(h + 1) * DH]
            kh = k[h * DH:(h + 1) * DH]
            vh = v[h * DH:(h + 1) * DH]
            sc = _mm(qh, kh, ((0,), (0,))) * (1.0 / (DH ** 0.5))
            sc = sc - jnp.max(sc, axis=1, keepdims=True)
            e = jnp.exp(sc)
            attn = e / jnp.sum(e, axis=1, keepdims=True)
            msgs.append(_mm(vh, attn, ((1,), (1,))))
        msg = jnp.concatenate(msgs, axis=0)
        msg = _mmnt(merge_w_ref[l], msg) + merge_b_ref[l]
        h1 = jnp.concatenate([desc, msg], axis=0)
        h1 = jnp.maximum(_mmnt(mlp_w1_ref[l], h1) + mlp_b1_ref[l], 0.0)
        desc = desc + _mmnt(mlp_w2_ref[l], h1) + mlp_b2_ref[l]
    d0 = _mmnt(fp_w_ref[...], desc) + fp_b_ref[...]
    desc0_out[...] = d0
    u_out[...] = _mm(d0, w1cat_ref[...], ((0,), (1,)))


def _gnn_tc(g, kpn, fm_w, fm_b, kw0, kb0, kw1, kb1, kw2, kb2, kw3, kb3,
            qkv_w, qkv_b, merge_w, merge_b, mlp_w1, mlp_b1, mlp_w2, mlp_b2,
            fp_w, fp_b, w1cat):
    return pl.pallas_call(
        _gnn_body,
        out_shape=[jax.ShapeDtypeStruct((D, N), F32),
                   jax.ShapeDtypeStruct((N, 9 * C1), F32)],
    )(g, kpn, fm_w, fm_b, kw0, kb0, kw1, kb1, kw2, kb2, kw3, kb3,
      qkv_w, qkv_b, merge_w, merge_b, mlp_w1, mlp_b1, mlp_w2, mlp_b2,
      fp_w, fp_b, w1cat)


def _conv2_body(f_ref, w2t_ref, cb2_ref, cb1_ref, bng_ref, bnb_ref,
                out_ref, a_sc, b_sc):
    i = pl.program_id(0)

    @pl.when(i == 0)
    def _stats():
        s = f_ref[:, PAD:PAD + HW]
        m1 = jnp.sum(s, axis=1, keepdims=True) * (1.0 / HW)
        m2 = jnp.sum(s * s, axis=1, keepdims=True) * (1.0 / HW)
        var = m2 - m1 * m1
        inv = lax.rsqrt(var + 1e-5)
        a = bng_ref[...] * inv
        a_sc[...] = a
        b_sc[...] = bnb_ref[...] - m1 * a

    a = a_sc[...]
    b = b_sc[...]
    w = f_ref[:, pl.ds(i * TP, WIN)]
    posg = lax.broadcasted_iota(jnp.int32, (1, WIN), 1) + (i * TP - PAD)
    validp = (posg >= 0) & (posg < HW)
    bnw = jnp.where(validp, jnp.maximum(w * a + b, 0.0), 0.0)
    col = lax.broadcasted_iota(jnp.int32, (1, TP), 1) % W
    parts = []
    for kh in range(3):
        for kw in range(3):
            delta = (kh - 1) * W + (kw - 1)
            sl = bnw[:, PAD + delta:PAD + delta + TP]
            if kw == 0:
                sl = jnp.where(col != 0, sl, 0.0)
            elif kw == 2:
                sl = jnp.where(col != W - 1, sl, 0.0)
            parts.append(sl.astype(BF16))
    cat = jnp.concatenate(parts, axis=0)
    acc = lax.dot_general(w2t_ref[...].astype(BF16), cat,
                          (((1,), (0,)), ((), ())),
                          preferred_element_type=F32)
    out_ref[...] = acc + cb2_ref[...]


def _conv2_tc(fcm, w2t, cb2c, cb1c, bngc, bnbc):
    return pl.pallas_call(
        _conv2_body,
        grid=(NT,),
        in_specs=[
            pl.BlockSpec((C1, FR), lambda i: (0, 0)),
            pl.BlockSpec((C2, 9 * C1), lambda i: (0, 0)),
            pl.BlockSpec((C2, 1), lambda i: (0, 0)),
            pl.BlockSpec((C1, 1), lambda i: (0, 0)),
            pl.BlockSpec((C1, 1), lambda i: (0, 0)),
            pl.BlockSpec((C1, 1), lambda i: (0, 0)),
        ],
        out_specs=pl.BlockSpec((C2, TP), lambda i: (0, i)),
        out_shape=jax.ShapeDtypeStruct((C2, HW), F32),
        scratch_shapes=[pltpu.VMEM((C1, 1), F32), pltpu.VMEM((C1, 1), F32)],
    )(fcm, w2t, cb2c, cb1c, bngc, bnbc)


def kernel(feature_2d, fm_w, fm_b, kw0, kb0, kw1, kb1, kw2, kb2, kw3, kb3,
           qkv_w, qkv_b, merge_w, merge_b, mlp_w1, mlp_b1, mlp_w2, mlp_b2,
           fp_w, fp_b, cw1, cb1, bn_g, bn_b, cw2, cb2, keypoints):
    c = feature_2d.shape[1]
    kr = keypoints[0, :, 0].astype(jnp.int32)
    kc = keypoints[0, :, 1].astype(jnp.int32)
    idx = kr * W + kc

    ft = feature_2d[0].reshape(c, HW).T
    g = _gather_sc(ft, idx)

    kpn = jnp.zeros((8, N), F32)
    kpn = kpn.at[0].set(kr.astype(F32) / H - 0.5)
    kpn = kpn.at[1].set(kc.astype(F32) / W - 0.5)

    w1cat = jnp.transpose(cw1, (2, 3, 0, 1)).reshape(9 * C1, D)

    desc0, u = _gnn_tc(
        g, kpn, fm_w, fm_b[:, None],
        kw0, kb0[:, None], kw1, kb1[:, None], kw2, kb2[:, None],
        kw3, kb3[:, None], qkv_w, qkv_b[..., None], merge_w,
        merge_b[..., None], mlp_w1, mlp_b1[..., None], mlp_w2,
        mlp_b2[..., None], fp_w, fp_b[:, None], w1cat)

    key = idx * 2048 + jnp.arange(N, dtype=jnp.int32)
    order = jnp.argsort(key)
    sidx = idx[order]
    win_sorted = jnp.concatenate([sidx[:-1] != sidx[1:],
                                  jnp.ones((1,), bool)])
    winner = jnp.zeros((N,), bool).at[order].set(win_sorted)

    t = jnp.arange(9, dtype=jnp.int32)
    dr = t // 3 - 1
    dc = t % 3 - 1
    orow = kr[:, None] - dr[None, :]
    ocol = kc[:, None] - dc[None, :]
    valid = ((orow >= 0) & (orow < H) & (ocol >= 0) & (ocol < W)
             & winner[:, None])
    posf = jnp.where(valid, PAD + orow * W + ocol, -1).reshape(NTAP)

    uf = u.reshape(N, 9, NC, 32).transpose(2, 0, 1, 3).reshape(NC, NTAP, 32)
    q = jnp.where(posf >= 0, posf % 4, 0)
    slot = lax.broadcasted_iota(jnp.int32, (NTAP, 128), 1) // 32
    lanemask = (slot == q[:, None]).astype(F32)
    u4 = jnp.tile(uf, (1, 1, 4)) * lanemask[None]

    prow = posf // 4
    pos3 = [jnp.where((posf >= 0) & (prow >= p * PR) & (prow < (p + 1) * PR),
                      prow - p * PR, PR).astype(jnp.int32)
            for p in range(PASSES)]
    zr = jnp.zeros((STRIPE, 128), F32)
    f4 = _scatter_sc(u4, pos3, zr)
    fcm = jnp.transpose(f4.reshape(NC, RTOT, 4, 32),
                        (0, 3, 1, 2)).reshape(C1, FR)

    w2t = jnp.transpose(cw2, (0, 2, 3, 1)).reshape(C2, 9 * C1)
    out = _conv2_tc(fcm, w2t, cb2[:, None], cb1[:, None],
                    bn_g[:, None], bn_b[:, None])

    y = out.reshape(1, C2, H, W)
    return (y, desc0[None])
